# baseline (speedup 1.0000x reference)
import functools

import jax
import jax.numpy as jnp
from jax import lax
from jax.experimental import pallas as pl
from jax.experimental.pallas import tpu as pltpu
from jax.experimental.pallas import tpu_sc as plsc

_EMBED = 128
_HALF = 64
_MAXD2 = 256
_B = 1024 * 200
_NW = 32
_PER_W = _B // _NW
_CHUNK = 256
_NCHUNK = _PER_W // _CHUNK
_NBUF = 2
_G = 1

_mesh = plsc.VectorSubcoreMesh(core_axis_name="c", subcore_axis_name="s")


@functools.partial(
    pl.kernel,
    out_type=jax.ShapeDtypeStruct((2 * _B, _HALF), jnp.float32),
    mesh=_mesh,
    compiler_params=pltpu.CompilerParams(use_tc_tiling_on_sc=False),
    scratch_types=[
        pltpu.VMEM((_PER_W,), jnp.int32),
        pltpu.VMEM((_PER_W,), jnp.int32),
        pltpu.VMEM((2 * _PER_W,), jnp.int32),
        pltpu.VMEM((_NBUF, 2 * _CHUNK, _HALF), jnp.float32),
        pltpu.VMEM((16,), jnp.int32),
        pltpu.VMEM((16, _EMBED), jnp.float32),
        pltpu.VMEM((16, _EMBED), jnp.float32),
        pltpu.VMEM((16, _HALF), jnp.float32),
        pltpu.VMEM((16, _HALF), jnp.float32),
        pltpu.VMEM_SHARED((2 * _MAXD2, _HALF), jnp.float32),
        pltpu.SemaphoreType.DMA((_NBUF,)),
        pltpu.SemaphoreType.DMA((_NBUF,)),
        pltpu.SemaphoreType.DMA,
    ],
)
def _gather_kernel(d1_hbm, d2_hbm, table_hbm, out_hbm,
                   d1_v, d2_v, idx_v, rows_v,
                   hidx_v, tmph_v, tmpw_v, ch_v, cw_v, cop_sh,
                   sem_g, sem_s, sem_t):
    sid = lax.axis_index("s")
    wid = sid * 2 + lax.axis_index("c")
    base = wid * _PER_W
    iota16 = lax.broadcasted_iota(jnp.int32, (16,), 0)

    hidx_v[...] = (iota16 + sid * 16) * _MAXD2
    pltpu.async_copy(table_hbm.at[hidx_v], tmph_v, sem_t).wait()
    pltpu.sync_copy(table_hbm.at[pl.ds(sid * 16, 16)], tmpw_v)
    for r in range(16):
        for k in range(_HALF // 16):
            s = pl.ds(k * 16, 16)
            cw_v[r, s] = tmpw_v[r, s]
            ch_v[r, s] = tmph_v[r, pl.ds(_HALF + k * 16, 16)]
    pltpu.sync_copy(cw_v, cop_sh.at[pl.ds(sid * 16, 16)])
    pltpu.sync_copy(ch_v, cop_sh.at[pl.ds(_MAXD2 + sid * 16, 16)])

    pltpu.sync_copy(d1_hbm.at[pl.ds(base, _PER_W)], d1_v)
    pltpu.sync_copy(d2_hbm.at[pl.ds(base, _PER_W)], d2_v)
    plsc.subcore_barrier()

    even_lane = (iota16 & 1) == 0
    pair_lo = lax.shift_right_logical(iota16, 1)
    pair_hi = pair_lo + 8

    def _lane_gather(v, i):
        return lax.gather(
            v, lax.broadcast_in_dim(i, (16, 1), (0,)),
            lax.GatherDimensionNumbers(offset_dims=(),
                                       collapsed_slice_dims=(0,),
                                       start_index_map=(0,)),
            (1,), mode=lax.GatherScatterMode.PROMISE_IN_BOUNDS)

    def compute_idx(c):
        for k in range(_CHUNK // 16):
            s = pl.ds(c * _CHUNK + k * 16, 16)
            h16 = d2_v[s]
            w16 = d1_v[s] + _MAXD2
            pos = (c * _CHUNK + k * 16) * 2
            for half, sel in ((pair_lo, 0), (pair_hi, 16)):
                mix = jnp.where(even_lane,
                                _lane_gather(h16, half),
                                _lane_gather(w16, half))
                idx_v[pl.ds(pos + sel, 16)] = mix

    def gather(c, buf):
        pltpu.async_copy(
            cop_sh.at[idx_v.at[pl.ds(c * 2 * _CHUNK, 2 * _CHUNK)]],
            rows_v.at[buf], sem_g.at[buf])

    for j in range(_G):
        compute_idx(j)
        gather(j, j)

    @pl.loop(0, _NCHUNK)
    def _chunk(j):
        b = lax.rem(j, _NBUF)

        @pl.when(j < _NCHUNK - _G)
        def _prefetch():
            nb = lax.rem(j + _G, _NBUF)
            compute_idx(j + _G)

            @pl.when(j >= _NBUF - _G)
            def _wait_old_store():
                pltpu.make_async_copy(
                    rows_v.at[nb],
                    out_hbm.at[pl.ds(2 * base, 2 * _CHUNK)],
                    sem_s.at[nb],
                ).wait()
            gather(j + _G, nb)

        pltpu.make_async_copy(
            cop_sh.at[idx_v.at[pl.ds(j * 2 * _CHUNK, 2 * _CHUNK)]],
            rows_v.at[b], sem_g.at[b]).wait()
        pltpu.async_copy(
            rows_v.at[b],
            out_hbm.at[pl.ds(2 * (base + j * _CHUNK), 2 * _CHUNK)],
            sem_s.at[b],
        )

    for t in range(_NCHUNK - _NBUF + _G, _NCHUNK):
        pltpu.make_async_copy(
            rows_v.at[t % _NBUF],
            out_hbm.at[pl.ds(2 * base, 2 * _CHUNK)],
            sem_s.at[t % _NBUF],
        ).wait()


def kernel(dim1_indices, dim2_indices, pos_embed):
    d1 = dim1_indices.reshape(-1)
    d2 = dim2_indices.reshape(-1)
    out = _gather_kernel(d1, d2, pos_embed)
    return out.reshape(dim1_indices.shape + (pos_embed.shape[1],))


# speedup vs baseline: 1.1522x; 1.1522x over previous
import functools

import jax
import jax.numpy as jnp
from jax import lax
from jax.experimental import pallas as pl
from jax.experimental.pallas import tpu as pltpu
from jax.experimental.pallas import tpu_sc as plsc

_EMBED = 128
_HALF = 64
_MAXD2 = 256
_B = 1024 * 200
_NW = 32
_PER_W = _B // _NW
_CHUNK = 128
_NCHUNK = _PER_W // _CHUNK
_NBUF = 5
_G = 2

_mesh = plsc.VectorSubcoreMesh(core_axis_name="c", subcore_axis_name="s")


@functools.partial(
    pl.kernel,
    out_type=jax.ShapeDtypeStruct((2 * _B, _HALF), jnp.float32),
    mesh=_mesh,
    compiler_params=pltpu.CompilerParams(use_tc_tiling_on_sc=False),
    scratch_types=[
        pltpu.VMEM((_PER_W,), jnp.int32),
        pltpu.VMEM((_PER_W,), jnp.int32),
        pltpu.VMEM((2 * _PER_W,), jnp.int32),
        pltpu.VMEM((_NBUF, 2 * _CHUNK, _HALF), jnp.float32),
        pltpu.VMEM((16,), jnp.int32),
        pltpu.VMEM((16, _EMBED), jnp.float32),
        pltpu.VMEM((16, _EMBED), jnp.float32),
        pltpu.VMEM((16, _HALF), jnp.float32),
        pltpu.VMEM((16, _HALF), jnp.float32),
        pltpu.VMEM_SHARED((2 * _MAXD2, _HALF), jnp.float32),
        pltpu.SemaphoreType.DMA((_NBUF,)),
        pltpu.SemaphoreType.DMA((_NBUF,)),
        pltpu.SemaphoreType.DMA,
    ],
)
def _gather_kernel(d1_hbm, d2_hbm, table_hbm, out_hbm,
                   d1_v, d2_v, idx_v, rows_v,
                   hidx_v, tmph_v, tmpw_v, ch_v, cw_v, cop_sh,
                   sem_g, sem_s, sem_t):
    sid = lax.axis_index("s")
    wid = sid * 2 + lax.axis_index("c")
    base = wid * _PER_W
    iota16 = lax.broadcasted_iota(jnp.int32, (16,), 0)

    hidx_v[...] = (iota16 + sid * 16) * _MAXD2
    pltpu.async_copy(table_hbm.at[hidx_v], tmph_v, sem_t).wait()
    pltpu.sync_copy(table_hbm.at[pl.ds(sid * 16, 16)], tmpw_v)
    for r in range(16):
        for k in range(_HALF // 16):
            s = pl.ds(k * 16, 16)
            cw_v[r, s] = tmpw_v[r, s]
            ch_v[r, s] = tmph_v[r, pl.ds(_HALF + k * 16, 16)]
    pltpu.sync_copy(cw_v, cop_sh.at[pl.ds(sid * 16, 16)])
    pltpu.sync_copy(ch_v, cop_sh.at[pl.ds(_MAXD2 + sid * 16, 16)])

    pltpu.sync_copy(d1_hbm.at[pl.ds(base, _PER_W)], d1_v)
    pltpu.sync_copy(d2_hbm.at[pl.ds(base, _PER_W)], d2_v)
    plsc.subcore_barrier()

    even_lane = (iota16 & 1) == 0
    pair_lo = lax.shift_right_logical(iota16, 1)
    pair_hi = pair_lo + 8

    def _lane_gather(v, i):
        return lax.gather(
            v, lax.broadcast_in_dim(i, (16, 1), (0,)),
            lax.GatherDimensionNumbers(offset_dims=(),
                                       collapsed_slice_dims=(0,),
                                       start_index_map=(0,)),
            (1,), mode=lax.GatherScatterMode.PROMISE_IN_BOUNDS)

    def compute_idx(c):
        for k in range(_CHUNK // 16):
            s = pl.ds(c * _CHUNK + k * 16, 16)
            h16 = d2_v[s]
            w16 = d1_v[s] + _MAXD2
            pos = (c * _CHUNK + k * 16) * 2
            for half, sel in ((pair_lo, 0), (pair_hi, 16)):
                mix = jnp.where(even_lane,
                                _lane_gather(h16, half),
                                _lane_gather(w16, half))
                idx_v[pl.ds(pos + sel, 16)] = mix

    def gather(c, buf):
        pltpu.async_copy(
            cop_sh.at[idx_v.at[pl.ds(c * 2 * _CHUNK, 2 * _CHUNK)]],
            rows_v.at[buf], sem_g.at[buf])

    for j in range(_G):
        compute_idx(j)
        gather(j, j)

    @pl.loop(0, _NCHUNK)
    def _chunk(j):
        b = lax.rem(j, _NBUF)

        @pl.when(j < _NCHUNK - _G)
        def _prefetch():
            nb = lax.rem(j + _G, _NBUF)
            compute_idx(j + _G)

            @pl.when(j >= _NBUF - _G)
            def _wait_old_store():
                pltpu.make_async_copy(
                    rows_v.at[nb],
                    out_hbm.at[pl.ds(2 * base, 2 * _CHUNK)],
                    sem_s.at[nb],
                ).wait()
            gather(j + _G, nb)

        pltpu.make_async_copy(
            cop_sh.at[idx_v.at[pl.ds(j * 2 * _CHUNK, 2 * _CHUNK)]],
            rows_v.at[b], sem_g.at[b]).wait()
        pltpu.async_copy(
            rows_v.at[b],
            out_hbm.at[pl.ds(2 * (base + j * _CHUNK), 2 * _CHUNK)],
            sem_s.at[b],
        )

    for t in range(_NCHUNK - _NBUF + _G, _NCHUNK):
        pltpu.make_async_copy(
            rows_v.at[t % _NBUF],
            out_hbm.at[pl.ds(2 * base, 2 * _CHUNK)],
            sem_s.at[t % _NBUF],
        ).wait()


def kernel(dim1_indices, dim2_indices, pos_embed):
    d1 = dim1_indices.reshape(-1)
    d2 = dim2_indices.reshape(-1)
    out = _gather_kernel(d1, d2, pos_embed)
    return out.reshape(dim1_indices.shape + (pos_embed.shape[1],))


# speedup vs baseline: 1.1559x; 1.0032x over previous
import functools

import jax
import jax.numpy as jnp
from jax import lax
from jax.experimental import pallas as pl
from jax.experimental.pallas import tpu as pltpu
from jax.experimental.pallas import tpu_sc as plsc

_EMBED = 128
_HALF = 64
_MAXD2 = 256
_B = 1024 * 200
_NW = 32
_PER_W = _B // _NW
_CHUNK = 128
_NCHUNK = _PER_W // _CHUNK
_NBUF = 5
_G = 2

_mesh = plsc.VectorSubcoreMesh(core_axis_name="c", subcore_axis_name="s")


@functools.partial(
    pl.kernel,
    out_type=jax.ShapeDtypeStruct((2 * _B, _HALF), jnp.float32),
    mesh=_mesh,
    compiler_params=pltpu.CompilerParams(use_tc_tiling_on_sc=False, disable_bounds_checks=True, disable_semaphore_checks=True),
    scratch_types=[
        pltpu.VMEM((_PER_W,), jnp.int32),
        pltpu.VMEM((_PER_W,), jnp.int32),
        pltpu.VMEM((2 * _PER_W,), jnp.int32),
        pltpu.VMEM((_NBUF, 2 * _CHUNK, _HALF), jnp.float32),
        pltpu.VMEM((16,), jnp.int32),
        pltpu.VMEM((16, _EMBED), jnp.float32),
        pltpu.VMEM((16, _EMBED), jnp.float32),
        pltpu.VMEM((16, _HALF), jnp.float32),
        pltpu.VMEM((16, _HALF), jnp.float32),
        pltpu.VMEM_SHARED((2 * _MAXD2, _HALF), jnp.float32),
        pltpu.SemaphoreType.DMA((_NBUF,)),
        pltpu.SemaphoreType.DMA((_NBUF,)),
        pltpu.SemaphoreType.DMA,
    ],
)
def _gather_kernel(d1_hbm, d2_hbm, table_hbm, out_hbm,
                   d1_v, d2_v, idx_v, rows_v,
                   hidx_v, tmph_v, tmpw_v, ch_v, cw_v, cop_sh,
                   sem_g, sem_s, sem_t):
    sid = lax.axis_index("s")
    wid = sid * 2 + lax.axis_index("c")
    base = wid * _PER_W
    iota16 = lax.broadcasted_iota(jnp.int32, (16,), 0)

    hidx_v[...] = (iota16 + sid * 16) * _MAXD2
    pltpu.async_copy(table_hbm.at[hidx_v], tmph_v, sem_t).wait()
    pltpu.sync_copy(table_hbm.at[pl.ds(sid * 16, 16)], tmpw_v)
    for r in range(16):
        for k in range(_HALF // 16):
            s = pl.ds(k * 16, 16)
            cw_v[r, s] = tmpw_v[r, s]
            ch_v[r, s] = tmph_v[r, pl.ds(_HALF + k * 16, 16)]
    pltpu.sync_copy(cw_v, cop_sh.at[pl.ds(sid * 16, 16)])
    pltpu.sync_copy(ch_v, cop_sh.at[pl.ds(_MAXD2 + sid * 16, 16)])

    pltpu.sync_copy(d1_hbm.at[pl.ds(base, _PER_W)], d1_v)
    pltpu.sync_copy(d2_hbm.at[pl.ds(base, _PER_W)], d2_v)
    plsc.subcore_barrier()

    even_lane = (iota16 & 1) == 0
    pair_lo = lax.shift_right_logical(iota16, 1)
    pair_hi = pair_lo + 8

    def _lane_gather(v, i):
        return lax.gather(
            v, lax.broadcast_in_dim(i, (16, 1), (0,)),
            lax.GatherDimensionNumbers(offset_dims=(),
                                       collapsed_slice_dims=(0,),
                                       start_index_map=(0,)),
            (1,), mode=lax.GatherScatterMode.PROMISE_IN_BOUNDS)

    def compute_idx(c):
        for k in range(_CHUNK // 16):
            s = pl.ds(c * _CHUNK + k * 16, 16)
            h16 = d2_v[s]
            w16 = d1_v[s] + _MAXD2
            pos = (c * _CHUNK + k * 16) * 2
            for half, sel in ((pair_lo, 0), (pair_hi, 16)):
                mix = jnp.where(even_lane,
                                _lane_gather(h16, half),
                                _lane_gather(w16, half))
                idx_v[pl.ds(pos + sel, 16)] = mix

    def gather(c, buf):
        pltpu.async_copy(
            cop_sh.at[idx_v.at[pl.ds(c * 2 * _CHUNK, 2 * _CHUNK)]],
            rows_v.at[buf], sem_g.at[buf])

    for j in range(_G):
        compute_idx(j)
        gather(j, j)

    @pl.loop(0, _NCHUNK)
    def _chunk(j):
        b = lax.rem(j, _NBUF)

        @pl.when(j < _NCHUNK - _G)
        def _prefetch():
            nb = lax.rem(j + _G, _NBUF)
            compute_idx(j + _G)

            @pl.when(j >= _NBUF - _G)
            def _wait_old_store():
                pltpu.make_async_copy(
                    rows_v.at[nb],
                    out_hbm.at[pl.ds(2 * base, 2 * _CHUNK)],
                    sem_s.at[nb],
                ).wait()
            gather(j + _G, nb)

        pltpu.make_async_copy(
            cop_sh.at[idx_v.at[pl.ds(j * 2 * _CHUNK, 2 * _CHUNK)]],
            rows_v.at[b], sem_g.at[b]).wait()
        pltpu.async_copy(
            rows_v.at[b],
            out_hbm.at[pl.ds(2 * (base + j * _CHUNK), 2 * _CHUNK)],
            sem_s.at[b],
        )

    for t in range(_NCHUNK - _NBUF + _G, _NCHUNK):
        pltpu.make_async_copy(
            rows_v.at[t % _NBUF],
            out_hbm.at[pl.ds(2 * base, 2 * _CHUNK)],
            sem_s.at[t % _NBUF],
        ).wait()


def kernel(dim1_indices, dim2_indices, pos_embed):
    d1 = dim1_indices.reshape(-1)
    d2 = dim2_indices.reshape(-1)
    out = _gather_kernel(d1, d2, pos_embed)
    return out.reshape(dim1_indices.shape + (pos_embed.shape[1],))
